# Initial kernel scaffold; baseline (speedup 1.0000x reference)
import functools

import jax
import jax.numpy as jnp
from jax import lax
from jax.experimental import pallas as pl
from jax.experimental.pallas import tpu as pltpu
from jax.experimental.pallas import tpu_sc as plsc

_C = 2000
_H = 32
_E = 32
_B = 64
_SEQ = 6
_NSTEP = _SEQ - 1
_NIDX = _NSTEP * _B
_PAD = 384
_CT = 200
_NT = _C // _CT


def _sc_gather_body(gtab, etab, idxg, idxe, gr_out, em_out,
                    idxg_v, idxe_v, rows_v, erows_v, sem):
    nc = plsc.get_sparse_core_info().num_cores
    wid = lax.axis_index("s") * nc + lax.axis_index("c")
    base = wid * 24
    pltpu.sync_copy(idxg.at[pl.ds(base, 24)], idxg_v)
    pltpu.async_copy(gtab.at[idxg_v], rows_v, sem).wait()
    pltpu.sync_copy(rows_v, gr_out.at[pl.ds(base, 24)])
    pltpu.sync_copy(idxe.at[pl.ds(base, 24)], idxe_v)
    pltpu.async_copy(etab.at[idxe_v], erows_v, sem).wait()
    pltpu.sync_copy(erows_v, em_out.at[pl.ds(base, 24)])


def _sc_gather(gtab, etab, idxg, idxe):
    mesh = plsc.VectorSubcoreMesh(core_axis_name="c", subcore_axis_name="s")
    fn = functools.partial(
        pl.kernel,
        mesh=mesh,
        out_type=[
            jax.ShapeDtypeStruct((2 * _PAD, _C), jnp.float32),
            jax.ShapeDtypeStruct((2 * _PAD, _E), jnp.float32),
        ],
        scratch_types=[
            pltpu.VMEM((24,), jnp.int32),
            pltpu.VMEM((24,), jnp.int32),
            pltpu.VMEM((24, _C), jnp.float32),
            pltpu.VMEM((24, _E), jnp.float32),
            pltpu.SemaphoreType.DMA,
        ],
    )(_sc_gather_body)
    return fn(gtab, etab, idxg, idxe)


def _tc_body(qv, re_all, ecq_all, adj_all, rev_all, emb_c, eaw,
             A01, b01, L1ht, V01, W2bd, b2c, g01, bt01,
             WEA, bEA, Wg, bg,
             fsW1t, fsb1, fsW2t, fsb2, fsg, fsbt,
             wp, bpv, out_ref, ht_s, ec_s):
    f32 = jnp.float32
    ht_s[...] = jnp.zeros((_B, _C, _H), f32)
    ec_s[...] = jnp.dot(emb_c[:_C, :], V01[...],
                        preferred_element_type=f32)

    inv_n = f32(1.0 / (_B * _C))

    for i in range(_NSTEP):
        qt = qv[:, i:i + 1]
        qn = qv[:, i + 1:i + 2]
        re = re_all[i * _B:(i + 1) * _B, :]
        ecq = ecq_all[i * _B:(i + 1) * _B, :]

        def _gather_self(t, acc):
            ts = t * _CT
            htt = ht_s[:, pl.ds(ts, _CT), :]
            cg = ts + lax.broadcasted_iota(jnp.int32, (_B, _CT), 1)
            m = (cg == qt)[:, :, None]
            return acc + jnp.sum(jnp.where(m, htt, f32(0.0)), axis=1)

        selfh = lax.fori_loop(0, _NT, _gather_self,
                              jnp.zeros((_B, _H), f32))
        self_ht = jnp.concatenate([selfh, re], axis=1)
        s01 = jnp.dot(self_ht, A01[...],
                      preferred_element_type=f32) + b01[...]
        d01 = jnp.dot(re - ecq, V01[...],
                      preferred_element_type=f32)

        h1 = jnp.maximum(jnp.dot(self_ht, fsW1t[...],
                                 preferred_element_type=f32) + fsb1[...], 0.0)
        h2 = jnp.maximum(jnp.dot(h1, fsW2t[...],
                                 preferred_element_type=f32) + fsb2[...], 0.0)
        mu = jnp.mean(h2, axis=0, keepdims=True)
        va = jnp.mean((h2 - mu) * (h2 - mu), axis=0, keepdims=True)
        sf = (h2 - mu) * jax.lax.rsqrt(va + 1e-5) * fsg[...] + fsbt[...]

        def _u_tile(ts):
            htt = ht_s[:, pl.ds(ts, _CT), :]
            htt2 = htt.reshape(_B * _CT, _H)
            cg = ts + lax.broadcasted_iota(jnp.int32, (_B, _CT), 1)
            m = cg == qt
            z = (jnp.dot(htt2, L1ht[...], preferred_element_type=f32)
                 .reshape(_B, _CT, 2 * _H)
                 + s01[:, None, :]
                 + ec_s[pl.ds(ts, _CT), :][None, :, :]
                 + jnp.where(m[:, :, None], f32(1.0), f32(0.0)) * d01[:, None, :])
            a = jnp.maximum(z, 0.0).reshape(_B * _CT, 2 * _H)
            u = jnp.maximum(jnp.dot(a, W2bd[...],
                                    preferred_element_type=f32) + b2c[...], 0.0)
            return u, htt2, m, cg

        def _stats(t, carry):
            ssum, ssq = carry
            u, _, _, _ = _u_tile(t * _CT)
            ssum = ssum + jnp.sum(u, axis=0, keepdims=True)
            ssq = ssq + jnp.sum(u * u, axis=0, keepdims=True)
            return ssum, ssq

        ssum, ssq = lax.fori_loop(
            0, _NT, _stats,
            (jnp.zeros((1, 2 * _H), f32), jnp.zeros((1, 2 * _H), f32)))
        mu01 = ssum * inv_n
        var01 = ssq * inv_n - mu01 * mu01
        sc01 = g01[...] * jax.lax.rsqrt(var01 + 1e-5)
        sh01 = bt01[...] - mu01 * sc01

        def _update(t, pacc):
            ts = t * _CT
            u, htt2, m, cg = _u_tile(ts)
            n01 = u * sc01 + sh01
            n0 = n01[:, :_H].reshape(_B, _CT, _H)
            n1 = n01[:, _H:].reshape(_B, _CT, _H)
            adjt = adj_all[i * _B:(i + 1) * _B, pl.ds(ts, _CT)]
            revt = rev_all[i * _B:(i + 1) * _B, pl.ds(ts, _CT)]
            neigh = adjt[:, :, None] * n0 + revt[:, :, None] * n1
            mn = jnp.where(m[:, :, None], sf[:, None, :], neigh)
            e = (jnp.dot(mn.reshape(_B * _CT, _H), WEA[...],
                         preferred_element_type=f32) + bEA[...])
            eg = jax.nn.sigmoid(e[:, :_H]).reshape(_B, _CT, _H)
            ad = jnp.tanh(e[:, _H:]).reshape(_B, _CT, _H)
            w3 = eaw[pl.ds(ts, _CT), :][None, :, :]
            m2 = mn - w3 * eg * mn + w3 * ad
            X = jnp.concatenate([m2.reshape(_B * _CT, _H), htt2], axis=1)
            G = jnp.dot(X, Wg[...], preferred_element_type=f32) + bg[...]
            r = jax.nn.sigmoid(G[:, :_H])
            zg = jax.nn.sigmoid(G[:, _H:2 * _H])
            n = jnp.tanh(G[:, 2 * _H:3 * _H] + r * G[:, 3 * _H:])
            hn = (1.0 - zg) * n + zg * htt2
            hn3 = hn.reshape(_B, _CT, _H)
            ht_s[:, pl.ds(ts, _CT), :] = hn3
            pv = jnp.sum(hn3 * wp[...][None, :, :], axis=2)
            mnx = cg == qn
            return pacc + jnp.sum(jnp.where(mnx, pv, f32(0.0)),
                                  axis=1, keepdims=True)

        pacc = lax.fori_loop(0, _NT, _update, jnp.zeros((_B, 1), f32))
        out_ref[:, i:i + 1] = jax.nn.sigmoid(pacc + bpv[...])


def _tc_call(args):
    return pl.pallas_call(
        _tc_body,
        out_shape=jax.ShapeDtypeStruct((_B, _NSTEP), jnp.float32),
        scratch_shapes=[
            pltpu.VMEM((_B, _C, _H), jnp.float32),
            pltpu.VMEM((_C, 2 * _H), jnp.float32),
        ],
    )(*args)


def kernel(features, questions, emb_x, emb_c, graph, fs_W1, fs_b1, fs_W2,
           fs_b2, fs_g, fs_bt, fn0_W1, fn0_b1, fn0_W2, fn0_b2, fn0_g,
           fn0_bt, fn1_W1, fn1_b1, fn1_W2, fn1_b2, fn1_g, fn1_bt, ea_w,
           ea_We, ea_be, ea_Wa, ea_ba, gru_Wih, gru_Whh, gru_bih, gru_bhh,
           Wp, bp):
    f32 = jnp.float32
    qi = questions.astype(jnp.int32)
    xi = features.astype(jnp.int32)

    qt_flat = qi[:, :_NSTEP].T.reshape(_NIDX)
    xt_flat = xi[:, :_NSTEP].T.reshape(_NIDX)
    padz = jnp.zeros((_PAD - _NIDX,), jnp.int32)
    idxg = jnp.concatenate([qt_flat, padz, qt_flat + _C, padz])
    idxe = jnp.concatenate([xt_flat, padz, qt_flat + 2 * _C, padz])

    gtab = jnp.concatenate([graph, graph.T], axis=0)
    etab = jnp.concatenate([emb_x, emb_c[:_C, :]], axis=0)

    gr_out, em_out = _sc_gather(gtab, etab, idxg, idxe)
    adj_all = gr_out[:_NIDX]
    rev_all = gr_out[_PAD:_PAD + _NIDX]
    re_all = em_out[:_NIDX]
    ecq_all = em_out[_PAD:_PAD + _NIDX]

    W0t, W1t = fn0_W1.T, fn1_W1.T
    A01 = jnp.concatenate([W0t[:2 * _H], W1t[:2 * _H]], axis=1)
    L1ht = jnp.concatenate([W0t[2 * _H:3 * _H], W1t[2 * _H:3 * _H]], axis=1)
    V01 = jnp.concatenate([W0t[3 * _H:], W1t[3 * _H:]], axis=1)
    b01 = jnp.concatenate([fn0_b1, fn1_b1]).reshape(1, 2 * _H)
    zH = jnp.zeros((_H, _H), f32)
    W2bd = jnp.concatenate([
        jnp.concatenate([fn0_W2.T, zH], axis=1),
        jnp.concatenate([zH, fn1_W2.T], axis=1)], axis=0)
    b2c = jnp.concatenate([fn0_b2, fn1_b2]).reshape(1, 2 * _H)
    g01 = jnp.concatenate([fn0_g, fn1_g]).reshape(1, 2 * _H)
    bt01 = jnp.concatenate([fn0_bt, fn1_bt]).reshape(1, 2 * _H)
    WEA = jnp.concatenate([ea_We.T, ea_Wa.T], axis=1)
    bEA = jnp.concatenate([ea_be, ea_ba]).reshape(1, 2 * _H)
    WihT, WhhT = gru_Wih.T, gru_Whh.T
    Wg = jnp.concatenate([
        jnp.concatenate([WihT, zH], axis=1),
        jnp.concatenate([WhhT[:, :2 * _H], zH, WhhT[:, 2 * _H:]], axis=1),
    ], axis=0)
    bg = jnp.concatenate([
        gru_bih[:2 * _H] + gru_bhh[:2 * _H],
        gru_bih[2 * _H:], gru_bhh[2 * _H:]]).reshape(1, 4 * _H)

    args = (qi, re_all, ecq_all, adj_all, rev_all, emb_c,
            ea_w.reshape(_C, 1),
            A01, b01, L1ht, V01, W2bd, b2c, g01, bt01,
            WEA, bEA, Wg, bg,
            fs_W1.T, fs_b1.reshape(1, _H), fs_W2.T, fs_b2.reshape(1, _H),
            fs_g.reshape(1, _H), fs_bt.reshape(1, _H),
            Wp.reshape(1, _H), bp.reshape(1, 1))
    return _tc_call(args)


# speedup vs baseline: 3.6852x; 3.6852x over previous
import functools

import jax
import jax.numpy as jnp
from jax import lax
from jax.experimental import pallas as pl
from jax.experimental.pallas import tpu as pltpu
from jax.experimental.pallas import tpu_sc as plsc

_C = 2000
_H = 32
_E = 32
_B = 64
_SEQ = 6
_NSTEP = _SEQ - 1
_NIDX = _NSTEP * _B
_PAD = 384
_CW = 2048
_EW = 128


def _sc_gather_body(gtab, etab, idxg, idxe, gr_out, em_out,
                    idxg_v, idxe_v, rows_v, erows_v, sem):
    nc = plsc.get_sparse_core_info().num_cores
    wid = lax.axis_index("s") * nc + lax.axis_index("c")
    base = wid * 24
    pltpu.sync_copy(idxg.at[pl.ds(base, 24)], idxg_v)
    pltpu.async_copy(gtab.at[idxg_v], rows_v, sem).wait()
    pltpu.sync_copy(rows_v, gr_out.at[pl.ds(base, 24)])
    pltpu.sync_copy(idxe.at[pl.ds(base, 24)], idxe_v)
    pltpu.async_copy(etab.at[idxe_v], erows_v, sem).wait()
    pltpu.sync_copy(erows_v, em_out.at[pl.ds(base, 24)])


def _sc_gather(gtab, etab, idxg, idxe):
    mesh = plsc.VectorSubcoreMesh(core_axis_name="c", subcore_axis_name="s")
    fn = functools.partial(
        pl.kernel,
        mesh=mesh,
        out_type=[
            jax.ShapeDtypeStruct((2 * _PAD, _CW), jnp.float32),
            jax.ShapeDtypeStruct((2 * _PAD, _EW), jnp.float32),
        ],
        scratch_types=[
            pltpu.VMEM((24,), jnp.int32),
            pltpu.VMEM((24,), jnp.int32),
            pltpu.VMEM((24, _CW), jnp.float32),
            pltpu.VMEM((24, _EW), jnp.float32),
            pltpu.SemaphoreType.DMA,
        ],
    )(_sc_gather_body)
    return fn(gtab, etab, idxg, idxe)


def _tc_body(qs, reT, ecqT, adj_all, rev_all, embcT, eaw,
             WA, b01, WU, WV, W2T, b2c, g01, bt01,
             WEAo, bEA, WG, bg,
             fsW1, fsb1, fsW2, fsb2, fsg, fsbt,
             wp, bpv, out_ref, ht_s, ec_s):
    f32 = jnp.float32
    dot = functools.partial(jnp.dot, preferred_element_type=f32)
    ht_s[...] = jnp.zeros((_B * _H, _CW), f32)
    ec_s[...] = dot(WV[...], embcT[...])
    lane = lax.broadcasted_iota(jnp.int32, (1, _CW), 1)
    valid = (lane < _C).astype(f32)
    bcol = lax.broadcasted_iota(jnp.int32, (_B, 1), 0)
    brow = lax.broadcasted_iota(jnp.int32, (1, _B), 1)
    inv_n = f32(1.0 / (_B * _C))

    for i in range(_NSTEP):
        reTi = reT[:, i * _B:(i + 1) * _B]
        ecqTi = ecqT[:, i * _B:(i + 1) * _B]
        adjB = adj_all[i * _B:(i + 1) * _B, :]
        revB = rev_all[i * _B:(i + 1) * _B, :]

        def _l1(b, acc):
            X = ht_s[pl.ds(b * _H, _H), :]
            mq = (lane == qs[b, i]).astype(f32)
            col = jnp.sum(X * mq, axis=1, keepdims=True)
            oh = (brow == b).astype(f32)
            return acc + col * oh

        selfhT = lax.fori_loop(0, _B, _l1, jnp.zeros((_H, _B), f32))
        shT = jnp.concatenate([selfhT, reTi], axis=0)
        s01T = dot(WA[...], shT) + b01[...]
        d01T = dot(WV[...], reTi - ecqTi)

        h1 = jnp.maximum(dot(fsW1[...], shT) + fsb1[...], 0.0)
        h2 = jnp.maximum(dot(fsW2[...], h1) + fsb2[...], 0.0)
        mu = jnp.mean(h2, axis=1, keepdims=True)
        va = jnp.mean((h2 - mu) * (h2 - mu), axis=1, keepdims=True)
        sfT = (h2 - mu) * lax.rsqrt(va + 1e-5) * fsg[...] + fsbt[...]

        def _ucols(b):
            X = ht_s[pl.ds(b * _H, _H), :]
            ohc = (bcol == b).astype(f32)
            s_c = dot(s01T, ohc)
            d_c = dot(d01T, ohc)
            mq = (lane == qs[b, i]).astype(f32)
            zT = dot(WU[...], X) + s_c + ec_s[...] + d_c * mq
            uT = jnp.maximum(dot(W2T[...], jnp.maximum(zT, 0.0))
                             + b2c[...], 0.0) * valid
            return X, uT, mq, ohc

        def _l2(b, carry):
            ssum, ssq = carry
            _, uT, _, _ = _ucols(b)
            return (ssum + jnp.sum(uT, axis=1, keepdims=True),
                    ssq + jnp.sum(uT * uT, axis=1, keepdims=True))

        ssum, ssq = lax.fori_loop(
            0, _B, _l2,
            (jnp.zeros((2 * _H, 1), f32), jnp.zeros((2 * _H, 1), f32)))
        mu01 = ssum * inv_n
        var01 = ssq * inv_n - mu01 * mu01
        sc01 = g01[...] * lax.rsqrt(var01 + 1e-5)
        sh01 = bt01[...] - mu01 * sc01

        def _l3(b, pacc):
            X, uT, mq, ohc = _ucols(b)
            n01 = uT * sc01 + sh01
            n0 = n01[:_H, :]
            n1 = n01[_H:, :]
            ohr = (brow == b).astype(f32)
            r_adj = dot(ohr, adjB)
            r_rev = dot(ohr, revB)
            neigh = r_adj * n0 + r_rev * n1
            sf_c = dot(sfT, ohc)
            mn = neigh + mq * (sf_c - neigh)
            eT = dot(WEAo[...], mn) + bEA[...]
            eg = jax.nn.sigmoid(eT[:_H, :])
            ad = jnp.tanh(eT[_H:, :])
            m2 = mn - eaw[...] * eg * mn + eaw[...] * ad
            XT = jnp.concatenate([m2, X], axis=0)
            G = dot(WG[...], XT) + bg[...]
            r = jax.nn.sigmoid(G[:_H, :])
            zg = jax.nn.sigmoid(G[_H:2 * _H, :])
            n = jnp.tanh(G[2 * _H:3 * _H, :] + r * G[3 * _H:, :])
            hn = (1.0 - zg) * n + zg * X
            ht_s[pl.ds(b * _H, _H), :] = hn
            pv = jnp.sum(hn * wp[...], axis=0, keepdims=True)
            mqn = (lane == qs[b, i + 1]).astype(f32)
            scal = jnp.sum(pv * mqn, axis=1, keepdims=True)
            return pacc + scal * ohc

        pacc = lax.fori_loop(0, _B, _l3, jnp.zeros((_B, 1), f32))
        out_ref[:, i:i + 1] = jax.nn.sigmoid(pacc + bpv[...])


def _tc_call(args):
    return pl.pallas_call(
        _tc_body,
        out_shape=jax.ShapeDtypeStruct((_B, _NSTEP), jnp.float32),
        in_specs=[pl.BlockSpec(memory_space=pltpu.SMEM)]
        + [pl.BlockSpec(memory_space=pltpu.VMEM)] * 26,
        out_specs=pl.BlockSpec(memory_space=pltpu.VMEM),
        scratch_shapes=[
            pltpu.VMEM((_B * _H, _CW), jnp.float32),
            pltpu.VMEM((2 * _H, _CW), jnp.float32),
        ],
    )(*args)


def kernel(features, questions, emb_x, emb_c, graph, fs_W1, fs_b1, fs_W2,
           fs_b2, fs_g, fs_bt, fn0_W1, fn0_b1, fn0_W2, fn0_b2, fn0_g,
           fn0_bt, fn1_W1, fn1_b1, fn1_W2, fn1_b2, fn1_g, fn1_bt, ea_w,
           ea_We, ea_be, ea_Wa, ea_ba, gru_Wih, gru_Whh, gru_bih, gru_bhh,
           Wp, bp):
    f32 = jnp.float32
    qi = questions.astype(jnp.int32)
    xi = features.astype(jnp.int32)

    qt_flat = qi[:, :_NSTEP].T.reshape(_NIDX)
    xt_flat = xi[:, :_NSTEP].T.reshape(_NIDX)
    padz = jnp.zeros((_PAD - _NIDX,), jnp.int32)
    idxg = jnp.concatenate([qt_flat, padz, qt_flat + _C, padz])
    idxe = jnp.concatenate([xt_flat, padz, qt_flat + 2 * _C, padz])

    gtab = jnp.pad(jnp.concatenate([graph, graph.T], axis=0),
                   ((0, 0), (0, _CW - _C)))
    etab = jnp.pad(jnp.concatenate([emb_x, emb_c[:_C, :]], axis=0),
                   ((0, 0), (0, _EW - _E)))

    gr_out, em_out = _sc_gather(gtab, etab, idxg, idxe)
    adj_all = gr_out[:_NIDX]
    rev_all = gr_out[_PAD:_PAD + _NIDX]
    reT = em_out[:_NIDX, :_E].T
    ecqT = em_out[_PAD:_PAD + _NIDX, :_E].T

    WA = jnp.concatenate([fn0_W1[:, :2 * _H], fn1_W1[:, :2 * _H]], axis=0)
    WU = jnp.concatenate([fn0_W1[:, 2 * _H:3 * _H],
                          fn1_W1[:, 2 * _H:3 * _H]], axis=0)
    WV = jnp.concatenate([fn0_W1[:, 3 * _H:], fn1_W1[:, 3 * _H:]], axis=0)
    b01 = jnp.concatenate([fn0_b1, fn1_b1]).reshape(2 * _H, 1)
    zH = jnp.zeros((_H, _H), f32)
    W2T = jnp.concatenate([
        jnp.concatenate([fn0_W2, zH], axis=1),
        jnp.concatenate([zH, fn1_W2], axis=1)], axis=0)
    b2c = jnp.concatenate([fn0_b2, fn1_b2]).reshape(2 * _H, 1)
    g01 = jnp.concatenate([fn0_g, fn1_g]).reshape(2 * _H, 1)
    bt01 = jnp.concatenate([fn0_bt, fn1_bt]).reshape(2 * _H, 1)
    WEAo = jnp.concatenate([ea_We, ea_Wa], axis=0)
    bEA = jnp.concatenate([ea_be, ea_ba]).reshape(2 * _H, 1)
    zH2 = jnp.zeros((_H, _H), f32)
    WG = jnp.concatenate([
        jnp.concatenate([gru_Wih[:_H], gru_Whh[:_H]], axis=1),
        jnp.concatenate([gru_Wih[_H:2 * _H], gru_Whh[_H:2 * _H]], axis=1),
        jnp.concatenate([gru_Wih[2 * _H:], zH2], axis=1),
        jnp.concatenate([zH2, gru_Whh[2 * _H:]], axis=1),
    ], axis=0)
    bg = jnp.concatenate([
        gru_bih[:2 * _H] + gru_bhh[:2 * _H],
        gru_bih[2 * _H:], gru_bhh[2 * _H:]]).reshape(4 * _H, 1)

    embcT = jnp.pad(emb_c[:_C, :].T, ((0, 0), (0, _CW - _C)))
    eaw_r = jnp.pad(ea_w, (0, _CW - _C)).reshape(1, _CW)

    args = (qi, reT, ecqT, adj_all, rev_all, embcT, eaw_r,
            WA, b01, WU, WV, W2T, b2c, g01, bt01,
            WEAo, bEA, WG, bg,
            fs_W1, fs_b1.reshape(_H, 1), fs_W2, fs_b2.reshape(_H, 1),
            fs_g.reshape(_H, 1), fs_bt.reshape(_H, 1),
            Wp.reshape(_H, 1), bp.reshape(1, 1))
    return _tc_call(args)


# speedup vs baseline: 4.5980x; 1.2477x over previous
import functools

import jax
import jax.numpy as jnp
from jax import lax
from jax.experimental import pallas as pl
from jax.experimental.pallas import tpu as pltpu
from jax.experimental.pallas import tpu_sc as plsc

_C = 2000
_H = 32
_E = 32
_B = 64
_SEQ = 6
_NSTEP = _SEQ - 1
_NIDX = _NSTEP * _B
_PAD = 384
_CW = 2048
_EW = 128


def _sc_gather_body(gtab, etab, idxg, idxe, gr_out, em_out,
                    idxg_v, idxe_v, rows_v, erows_v, sem):
    nc = plsc.get_sparse_core_info().num_cores
    wid = lax.axis_index("s") * nc + lax.axis_index("c")
    base = wid * 24
    pltpu.sync_copy(idxg.at[pl.ds(base, 24)], idxg_v)
    pltpu.async_copy(gtab.at[idxg_v], rows_v, sem).wait()
    pltpu.sync_copy(rows_v, gr_out.at[pl.ds(base, 24)])
    pltpu.sync_copy(idxe.at[pl.ds(base, 24)], idxe_v)
    pltpu.async_copy(etab.at[idxe_v], erows_v, sem).wait()
    pltpu.sync_copy(erows_v, em_out.at[pl.ds(base, 24)])


def _sc_gather(gtab, etab, idxg, idxe):
    mesh = plsc.VectorSubcoreMesh(core_axis_name="c", subcore_axis_name="s")
    fn = functools.partial(
        pl.kernel,
        mesh=mesh,
        out_type=[
            jax.ShapeDtypeStruct((2 * _PAD, _CW), jnp.float32),
            jax.ShapeDtypeStruct((2 * _PAD, _EW), jnp.float32),
        ],
        scratch_types=[
            pltpu.VMEM((24,), jnp.int32),
            pltpu.VMEM((24,), jnp.int32),
            pltpu.VMEM((24, _CW), jnp.float32),
            pltpu.VMEM((24, _EW), jnp.float32),
            pltpu.SemaphoreType.DMA,
        ],
    )(_sc_gather_body)
    return fn(gtab, etab, idxg, idxe)


def _tc_body(qs, reT, ecqT, adj_all, rev_all, embcT, eaw,
             WA, b01, WU, WV, W2T, b2c, g01, bt01,
             WEAo, bEA, WG, bg,
             fsW1, fsb1, fsW2, fsb2, fsg, fsbt,
             wp, bpv, out_ref, ht_s, ec_s, u_cache):
    f32 = jnp.float32
    dot = functools.partial(jnp.dot, preferred_element_type=f32)
    ht_s[...] = jnp.zeros((_B * _H, _CW), f32)
    ec_s[...] = dot(WV[...], embcT[...])
    lane = lax.broadcasted_iota(jnp.int32, (1, _CW), 1)
    valid = (lane < _C).astype(f32)
    bcol = lax.broadcasted_iota(jnp.int32, (_B, 1), 0)
    brow = lax.broadcasted_iota(jnp.int32, (1, _B), 1)
    inv_n = f32(1.0 / (_B * _C))

    selfhT = jnp.zeros((_H, _B), f32)
    for i in range(_NSTEP):
        reTi = reT[:, i * _B:(i + 1) * _B]
        ecqTi = ecqT[:, i * _B:(i + 1) * _B]
        adjB = adj_all[i * _B:(i + 1) * _B, :]
        revB = rev_all[i * _B:(i + 1) * _B, :]

        shT = jnp.concatenate([selfhT, reTi], axis=0)
        s01T = dot(WA[...], shT) + b01[...]
        d01T = dot(WV[...], reTi - ecqTi)

        h1 = jnp.maximum(dot(fsW1[...], shT) + fsb1[...], 0.0)
        h2 = jnp.maximum(dot(fsW2[...], h1) + fsb2[...], 0.0)
        mu = jnp.mean(h2, axis=1, keepdims=True)
        va = jnp.mean((h2 - mu) * (h2 - mu), axis=1, keepdims=True)
        sfT = (h2 - mu) * lax.rsqrt(va + 1e-5) * fsg[...] + fsbt[...]

        def _l2(b, carry):
            ssum, ssq = carry
            X = ht_s[pl.ds(b * _H, _H), :]
            ohc = (bcol == b).astype(f32)
            s_c = dot(s01T, ohc)
            d_c = dot(d01T, ohc)
            mq = (lane == qs[b, i]).astype(f32)
            zT = dot(WU[...], X) + s_c + ec_s[...] + d_c * mq
            uT = jnp.maximum(dot(W2T[...], jnp.maximum(zT, 0.0))
                             + b2c[...], 0.0) * valid
            u_cache[pl.ds(b * 2 * _H, 2 * _H), :] = uT
            return (ssum + jnp.sum(uT, axis=1, keepdims=True),
                    ssq + jnp.sum(uT * uT, axis=1, keepdims=True))

        ssum, ssq = lax.fori_loop(
            0, _B, _l2,
            (jnp.zeros((2 * _H, 1), f32), jnp.zeros((2 * _H, 1), f32)))
        mu01 = ssum * inv_n
        var01 = ssq * inv_n - mu01 * mu01
        sc01 = g01[...] * lax.rsqrt(var01 + 1e-5)
        sh01 = bt01[...] - mu01 * sc01

        def _l3(b, carry):
            pacc, snext = carry
            X = ht_s[pl.ds(b * _H, _H), :]
            uT = u_cache[pl.ds(b * 2 * _H, 2 * _H), :]
            ohc = (bcol == b).astype(f32)
            mq = (lane == qs[b, i]).astype(f32)
            n01 = uT * sc01 + sh01
            n0 = n01[:_H, :]
            n1 = n01[_H:, :]
            ohr = (brow == b).astype(f32)
            r_adj = dot(ohr, adjB)
            r_rev = dot(ohr, revB)
            neigh = r_adj * n0 + r_rev * n1
            sf_c = dot(sfT, ohc)
            mn = neigh + mq * (sf_c - neigh)
            eT = dot(WEAo[...], mn) + bEA[...]
            eg = jax.nn.sigmoid(eT[:_H, :])
            ad = jnp.tanh(eT[_H:, :])
            m2 = mn - eaw[...] * eg * mn + eaw[...] * ad
            XT = jnp.concatenate([m2, X], axis=0)
            G = dot(WG[...], XT) + bg[...]
            r = jax.nn.sigmoid(G[:_H, :])
            zg = jax.nn.sigmoid(G[_H:2 * _H, :])
            n = jnp.tanh(G[2 * _H:3 * _H, :] + r * G[3 * _H:, :])
            hn = (1.0 - zg) * n + zg * X
            ht_s[pl.ds(b * _H, _H), :] = hn
            mqn = (lane == qs[b, i + 1]).astype(f32)
            hsel = hn * mqn
            col = jnp.sum(hsel, axis=1, keepdims=True)
            ohr = (brow == b).astype(f32)
            snext = snext + col * ohr
            pv = jnp.sum(col * wp[...], axis=0, keepdims=True)
            return pacc + pv * ohc, snext

        pacc, selfhT = lax.fori_loop(
            0, _B, _l3,
            (jnp.zeros((_B, 1), f32), jnp.zeros((_H, _B), f32)))
        out_ref[:, i:i + 1] = jax.nn.sigmoid(pacc + bpv[...])


def _tc_call(args):
    return pl.pallas_call(
        _tc_body,
        out_shape=jax.ShapeDtypeStruct((_B, _NSTEP), jnp.float32),
        in_specs=[pl.BlockSpec(memory_space=pltpu.SMEM)]
        + [pl.BlockSpec(memory_space=pltpu.VMEM)] * 26,
        out_specs=pl.BlockSpec(memory_space=pltpu.VMEM),
        scratch_shapes=[
            pltpu.VMEM((_B * _H, _CW), jnp.float32),
            pltpu.VMEM((2 * _H, _CW), jnp.float32),
            pltpu.VMEM((_B * 2 * _H, _CW), jnp.float32),
        ],
    )(*args)


def kernel(features, questions, emb_x, emb_c, graph, fs_W1, fs_b1, fs_W2,
           fs_b2, fs_g, fs_bt, fn0_W1, fn0_b1, fn0_W2, fn0_b2, fn0_g,
           fn0_bt, fn1_W1, fn1_b1, fn1_W2, fn1_b2, fn1_g, fn1_bt, ea_w,
           ea_We, ea_be, ea_Wa, ea_ba, gru_Wih, gru_Whh, gru_bih, gru_bhh,
           Wp, bp):
    f32 = jnp.float32
    qi = questions.astype(jnp.int32)
    xi = features.astype(jnp.int32)

    qt_flat = qi[:, :_NSTEP].T.reshape(_NIDX)
    xt_flat = xi[:, :_NSTEP].T.reshape(_NIDX)
    padz = jnp.zeros((_PAD - _NIDX,), jnp.int32)
    idxg = jnp.concatenate([qt_flat, padz, qt_flat + _C, padz])
    idxe = jnp.concatenate([xt_flat, padz, qt_flat + 2 * _C, padz])

    gtab = jnp.pad(jnp.concatenate([graph, graph.T], axis=0),
                   ((0, 0), (0, _CW - _C)))
    etab = jnp.pad(jnp.concatenate([emb_x, emb_c[:_C, :]], axis=0),
                   ((0, 0), (0, _EW - _E)))

    gr_out, em_out = _sc_gather(gtab, etab, idxg, idxe)
    adj_all = gr_out[:_NIDX]
    rev_all = gr_out[_PAD:_PAD + _NIDX]
    reT = em_out[:_NIDX, :_E].T
    ecqT = em_out[_PAD:_PAD + _NIDX, :_E].T

    WA = jnp.concatenate([fn0_W1[:, :2 * _H], fn1_W1[:, :2 * _H]], axis=0)
    WU = jnp.concatenate([fn0_W1[:, 2 * _H:3 * _H],
                          fn1_W1[:, 2 * _H:3 * _H]], axis=0)
    WV = jnp.concatenate([fn0_W1[:, 3 * _H:], fn1_W1[:, 3 * _H:]], axis=0)
    b01 = jnp.concatenate([fn0_b1, fn1_b1]).reshape(2 * _H, 1)
    zH = jnp.zeros((_H, _H), f32)
    W2T = jnp.concatenate([
        jnp.concatenate([fn0_W2, zH], axis=1),
        jnp.concatenate([zH, fn1_W2], axis=1)], axis=0)
    b2c = jnp.concatenate([fn0_b2, fn1_b2]).reshape(2 * _H, 1)
    g01 = jnp.concatenate([fn0_g, fn1_g]).reshape(2 * _H, 1)
    bt01 = jnp.concatenate([fn0_bt, fn1_bt]).reshape(2 * _H, 1)
    WEAo = jnp.concatenate([ea_We, ea_Wa], axis=0)
    bEA = jnp.concatenate([ea_be, ea_ba]).reshape(2 * _H, 1)
    zH2 = jnp.zeros((_H, _H), f32)
    WG = jnp.concatenate([
        jnp.concatenate([gru_Wih[:_H], gru_Whh[:_H]], axis=1),
        jnp.concatenate([gru_Wih[_H:2 * _H], gru_Whh[_H:2 * _H]], axis=1),
        jnp.concatenate([gru_Wih[2 * _H:], zH2], axis=1),
        jnp.concatenate([zH2, gru_Whh[2 * _H:]], axis=1),
    ], axis=0)
    bg = jnp.concatenate([
        gru_bih[:2 * _H] + gru_bhh[:2 * _H],
        gru_bih[2 * _H:], gru_bhh[2 * _H:]]).reshape(4 * _H, 1)

    embcT = jnp.pad(emb_c[:_C, :].T, ((0, 0), (0, _CW - _C)))
    eaw_r = jnp.pad(ea_w, (0, _CW - _C)).reshape(1, _CW)

    args = (qi, reT, ecqT, adj_all, rev_all, embcT, eaw_r,
            WA, b01, WU, WV, W2T, b2c, g01, bt01,
            WEAo, bEA, WG, bg,
            fs_W1, fs_b1.reshape(_H, 1), fs_W2, fs_b2.reshape(_H, 1),
            fs_g.reshape(_H, 1), fs_bt.reshape(_H, 1),
            Wp.reshape(_H, 1), bp.reshape(1, 1))
    return _tc_call(args)


# speedup vs baseline: 4.7405x; 1.0310x over previous
import functools

import jax
import jax.numpy as jnp
from jax import lax
from jax.experimental import pallas as pl
from jax.experimental.pallas import tpu as pltpu
from jax.experimental.pallas import tpu_sc as plsc

_C = 2000
_H = 32
_E = 32
_B = 64
_SEQ = 6
_NSTEP = _SEQ - 1
_NIDX = _NSTEP * _B
_PAD = 384
_CW = 2048
_EW = 128


def _sc_gather_body(gtab, etab, idxg, idxe, gr_out, em_out,
                    idxg_v, idxe_v, rows_v, erows_v, sem):
    nc = plsc.get_sparse_core_info().num_cores
    wid = lax.axis_index("s") * nc + lax.axis_index("c")
    base = wid * 24
    pltpu.sync_copy(idxg.at[pl.ds(base, 24)], idxg_v)
    pltpu.async_copy(gtab.at[idxg_v], rows_v, sem).wait()
    pltpu.sync_copy(rows_v, gr_out.at[pl.ds(base, 24)])
    pltpu.sync_copy(idxe.at[pl.ds(base, 24)], idxe_v)
    pltpu.async_copy(etab.at[idxe_v], erows_v, sem).wait()
    pltpu.sync_copy(erows_v, em_out.at[pl.ds(base, 24)])


def _sc_gather(gtab, etab, idxg, idxe):
    mesh = plsc.VectorSubcoreMesh(core_axis_name="c", subcore_axis_name="s")
    fn = functools.partial(
        pl.kernel,
        mesh=mesh,
        out_type=[
            jax.ShapeDtypeStruct((2 * _PAD, _CW), jnp.float32),
            jax.ShapeDtypeStruct((2 * _PAD, _EW), jnp.float32),
        ],
        scratch_types=[
            pltpu.VMEM((24,), jnp.int32),
            pltpu.VMEM((24,), jnp.int32),
            pltpu.VMEM((24, _CW), jnp.float32),
            pltpu.VMEM((24, _EW), jnp.float32),
            pltpu.SemaphoreType.DMA,
        ],
    )(_sc_gather_body)
    return fn(gtab, etab, idxg, idxe)


def _tc_body(qs, reT, ecqT, adj_all, rev_all, embcT, eaw,
             WA, b01, WU, WV, W2T, b2c, g01, bt01,
             WEAo, bEA, WG, bg,
             fsW1, fsb1, fsW2, fsb2, fsg, fsbt,
             wp, bpv, out_ref, ht_s, ec_s, u_cache):
    f32 = jnp.float32
    dot = functools.partial(jnp.dot, preferred_element_type=f32)
    ht_s[...] = jnp.zeros((_B * _H, _CW), f32)
    ec_s[...] = dot(WV[...], embcT[...])
    lane = lax.broadcasted_iota(jnp.int32, (1, _CW), 1)
    valid = (lane < _C).astype(f32)
    bcol = lax.broadcasted_iota(jnp.int32, (_B, 1), 0)
    brow = lax.broadcasted_iota(jnp.int32, (1, _B), 1)
    inv_n = f32(1.0 / (_B * _C))

    selfhT = jnp.zeros((_H, _B), f32)
    for i in range(_NSTEP):
        reTi = reT[:, i * _B:(i + 1) * _B]
        ecqTi = ecqT[:, i * _B:(i + 1) * _B]
        adjB = adj_all[i * _B:(i + 1) * _B, :]
        revB = rev_all[i * _B:(i + 1) * _B, :]

        shT = jnp.concatenate([selfhT, reTi], axis=0)
        s01T = dot(WA[...], shT) + b01[...]
        d01T = dot(WV[...], reTi - ecqTi)

        h1 = jnp.maximum(dot(fsW1[...], shT) + fsb1[...], 0.0)
        h2 = jnp.maximum(dot(fsW2[...], h1) + fsb2[...], 0.0)
        mu = jnp.mean(h2, axis=1, keepdims=True)
        va = jnp.mean((h2 - mu) * (h2 - mu), axis=1, keepdims=True)
        sfT = (h2 - mu) * lax.rsqrt(va + 1e-5) * fsg[...] + fsbt[...]

        def _l2(b, carry):
            ssum, ssq = carry
            X = ht_s[pl.ds(b * _H, _H), :]
            ohc = (bcol == b).astype(f32)
            s_c = dot(s01T, ohc)
            d_c = dot(d01T, ohc)
            mq = (lane == qs[b, i]).astype(f32)
            zT = dot(WU[...], X) + s_c + ec_s[...] + d_c * mq
            uT = jnp.maximum(dot(W2T[...], jnp.maximum(zT, 0.0))
                             + b2c[...], 0.0) * valid
            u_cache[pl.ds(b * 2 * _H, 2 * _H), :] = uT
            return (ssum + jnp.sum(uT, axis=1, keepdims=True),
                    ssq + jnp.sum(uT * uT, axis=1, keepdims=True))

        ssum, ssq = lax.fori_loop(
            0, _B, _l2,
            (jnp.zeros((2 * _H, 1), f32), jnp.zeros((2 * _H, 1), f32)))
        mu01 = ssum * inv_n
        var01 = ssq * inv_n - mu01 * mu01
        sc01 = g01[...] * lax.rsqrt(var01 + 1e-5)
        sh01 = bt01[...] - mu01 * sc01

        def _l3(b, carry):
            pacc, snext = carry
            X = ht_s[pl.ds(b * _H, _H), :]
            uT = u_cache[pl.ds(b * 2 * _H, 2 * _H), :]
            ohc = (bcol == b).astype(f32)
            mq = (lane == qs[b, i]).astype(f32)
            n01 = uT * sc01 + sh01
            n0 = n01[:_H, :]
            n1 = n01[_H:, :]
            r_adj = adj_all[pl.ds(i * _B + b, 1), :]
            r_rev = rev_all[pl.ds(i * _B + b, 1), :]
            neigh = r_adj * n0 + r_rev * n1
            sf_c = dot(sfT, ohc)
            mn = neigh + mq * (sf_c - neigh)
            eT = dot(WEAo[...], mn) + bEA[...]
            eg = jax.nn.sigmoid(eT[:_H, :])
            ad = jnp.tanh(eT[_H:, :])
            m2 = mn - eaw[...] * eg * mn + eaw[...] * ad
            XT = jnp.concatenate([m2, X], axis=0)
            G = dot(WG[...], XT) + bg[...]
            r = jax.nn.sigmoid(G[:_H, :])
            zg = jax.nn.sigmoid(G[_H:2 * _H, :])
            n = jnp.tanh(G[2 * _H:3 * _H, :] + r * G[3 * _H:, :])
            hn = (1.0 - zg) * n + zg * X
            ht_s[pl.ds(b * _H, _H), :] = hn
            mqn = (lane == qs[b, i + 1]).astype(f32)
            hsel = hn * mqn
            col = jnp.sum(hsel, axis=1, keepdims=True)
            ohr = (brow == b).astype(f32)
            snext = snext + col * ohr
            pv = jnp.sum(col * wp[...], axis=0, keepdims=True)
            return pacc + pv * ohc, snext

        pacc, selfhT = lax.fori_loop(
            0, _B, _l3,
            (jnp.zeros((_B, 1), f32), jnp.zeros((_H, _B), f32)))
        out_ref[:, i:i + 1] = jax.nn.sigmoid(pacc + bpv[...])


def _tc_call(args):
    return pl.pallas_call(
        _tc_body,
        out_shape=jax.ShapeDtypeStruct((_B, _NSTEP), jnp.float32),
        in_specs=[pl.BlockSpec(memory_space=pltpu.SMEM)]
        + [pl.BlockSpec(memory_space=pltpu.VMEM)] * 26,
        out_specs=pl.BlockSpec(memory_space=pltpu.VMEM),
        scratch_shapes=[
            pltpu.VMEM((_B * _H, _CW), jnp.float32),
            pltpu.VMEM((2 * _H, _CW), jnp.float32),
            pltpu.VMEM((_B * 2 * _H, _CW), jnp.float32),
        ],
    )(*args)


def kernel(features, questions, emb_x, emb_c, graph, fs_W1, fs_b1, fs_W2,
           fs_b2, fs_g, fs_bt, fn0_W1, fn0_b1, fn0_W2, fn0_b2, fn0_g,
           fn0_bt, fn1_W1, fn1_b1, fn1_W2, fn1_b2, fn1_g, fn1_bt, ea_w,
           ea_We, ea_be, ea_Wa, ea_ba, gru_Wih, gru_Whh, gru_bih, gru_bhh,
           Wp, bp):
    f32 = jnp.float32
    qi = questions.astype(jnp.int32)
    xi = features.astype(jnp.int32)

    qt_flat = qi[:, :_NSTEP].T.reshape(_NIDX)
    xt_flat = xi[:, :_NSTEP].T.reshape(_NIDX)
    padz = jnp.zeros((_PAD - _NIDX,), jnp.int32)
    idxg = jnp.concatenate([qt_flat, padz, qt_flat + _C, padz])
    idxe = jnp.concatenate([xt_flat, padz, qt_flat + 2 * _C, padz])

    gtab = jnp.pad(jnp.concatenate([graph, graph.T], axis=0),
                   ((0, 0), (0, _CW - _C)))
    etab = jnp.pad(jnp.concatenate([emb_x, emb_c[:_C, :]], axis=0),
                   ((0, 0), (0, _EW - _E)))

    gr_out, em_out = _sc_gather(gtab, etab, idxg, idxe)
    adj_all = gr_out[:_NIDX]
    rev_all = gr_out[_PAD:_PAD + _NIDX]
    reT = em_out[:_NIDX, :_E].T
    ecqT = em_out[_PAD:_PAD + _NIDX, :_E].T

    WA = jnp.concatenate([fn0_W1[:, :2 * _H], fn1_W1[:, :2 * _H]], axis=0)
    WU = jnp.concatenate([fn0_W1[:, 2 * _H:3 * _H],
                          fn1_W1[:, 2 * _H:3 * _H]], axis=0)
    WV = jnp.concatenate([fn0_W1[:, 3 * _H:], fn1_W1[:, 3 * _H:]], axis=0)
    b01 = jnp.concatenate([fn0_b1, fn1_b1]).reshape(2 * _H, 1)
    zH = jnp.zeros((_H, _H), f32)
    W2T = jnp.concatenate([
        jnp.concatenate([fn0_W2, zH], axis=1),
        jnp.concatenate([zH, fn1_W2], axis=1)], axis=0)
    b2c = jnp.concatenate([fn0_b2, fn1_b2]).reshape(2 * _H, 1)
    g01 = jnp.concatenate([fn0_g, fn1_g]).reshape(2 * _H, 1)
    bt01 = jnp.concatenate([fn0_bt, fn1_bt]).reshape(2 * _H, 1)
    WEAo = jnp.concatenate([ea_We, ea_Wa], axis=0)
    bEA = jnp.concatenate([ea_be, ea_ba]).reshape(2 * _H, 1)
    zH2 = jnp.zeros((_H, _H), f32)
    WG = jnp.concatenate([
        jnp.concatenate([gru_Wih[:_H], gru_Whh[:_H]], axis=1),
        jnp.concatenate([gru_Wih[_H:2 * _H], gru_Whh[_H:2 * _H]], axis=1),
        jnp.concatenate([gru_Wih[2 * _H:], zH2], axis=1),
        jnp.concatenate([zH2, gru_Whh[2 * _H:]], axis=1),
    ], axis=0)
    bg = jnp.concatenate([
        gru_bih[:2 * _H] + gru_bhh[:2 * _H],
        gru_bih[2 * _H:], gru_bhh[2 * _H:]]).reshape(4 * _H, 1)

    embcT = jnp.pad(emb_c[:_C, :].T, ((0, 0), (0, _CW - _C)))
    eaw_r = jnp.pad(ea_w, (0, _CW - _C)).reshape(1, _CW)

    args = (qi, reT, ecqT, adj_all, rev_all, embcT, eaw_r,
            WA, b01, WU, WV, W2T, b2c, g01, bt01,
            WEAo, bEA, WG, bg,
            fs_W1, fs_b1.reshape(_H, 1), fs_W2, fs_b2.reshape(_H, 1),
            fs_g.reshape(_H, 1), fs_bt.reshape(_H, 1),
            Wp.reshape(_H, 1), bp.reshape(1, 1))
    return _tc_call(args)


# speedup vs baseline: 4.7418x; 1.0003x over previous
import functools

import jax
import jax.numpy as jnp
from jax import lax
from jax.experimental import pallas as pl
from jax.experimental.pallas import tpu as pltpu
from jax.experimental.pallas import tpu_sc as plsc

_C = 2000
_H = 32
_E = 32
_B = 64
_SEQ = 6
_NSTEP = _SEQ - 1
_NIDX = _NSTEP * _B
_PAD = 384
_CW = 2048
_EW = 128


def _sc_gather_body(gtab, etab, idxg, idxe, gr_out, em_out,
                    idxg_v, idxe_v, rows_v, erows_v, sem):
    nc = plsc.get_sparse_core_info().num_cores
    wid = lax.axis_index("s") * nc + lax.axis_index("c")
    base = wid * 24
    pltpu.sync_copy(idxg.at[pl.ds(base, 24)], idxg_v)
    pltpu.async_copy(gtab.at[idxg_v], rows_v, sem).wait()
    pltpu.sync_copy(rows_v, gr_out.at[pl.ds(base, 24)])
    pltpu.sync_copy(idxe.at[pl.ds(base, 24)], idxe_v)
    pltpu.async_copy(etab.at[idxe_v], erows_v, sem).wait()
    pltpu.sync_copy(erows_v, em_out.at[pl.ds(base, 24)])


def _sc_gather(gtab, etab, idxg, idxe):
    mesh = plsc.VectorSubcoreMesh(core_axis_name="c", subcore_axis_name="s")
    fn = functools.partial(
        pl.kernel,
        mesh=mesh,
        out_type=[
            jax.ShapeDtypeStruct((2 * _PAD, _CW), jnp.float32),
            jax.ShapeDtypeStruct((2 * _PAD, _EW), jnp.float32),
        ],
        scratch_types=[
            pltpu.VMEM((24,), jnp.int32),
            pltpu.VMEM((24,), jnp.int32),
            pltpu.VMEM((24, _CW), jnp.float32),
            pltpu.VMEM((24, _EW), jnp.float32),
            pltpu.SemaphoreType.DMA,
        ],
    )(_sc_gather_body)
    return fn(gtab, etab, idxg, idxe)


def _tc_body(qs, reT, ecqT, adj_all, rev_all, embcT, eaw,
             WA, b01, WU, WV, W2T, b2c, g01, bt01,
             WEAo, bEA, WG, bg,
             fsW1, fsb1, fsW2, fsb2, fsg, fsbt,
             wp, bpv, out_ref, ht_s, ec_s, u_cache):
    f32 = jnp.float32
    dot = functools.partial(jnp.dot, preferred_element_type=f32)
    ht_s[...] = jnp.zeros((_B * _H, _CW), f32)
    ec_s[...] = dot(WV[...], embcT[...])
    lane = lax.broadcasted_iota(jnp.int32, (1, _CW), 1)
    valid = (lane < _C).astype(f32)
    bcol = lax.broadcasted_iota(jnp.int32, (_B, 1), 0)
    brow = lax.broadcasted_iota(jnp.int32, (1, _B), 1)
    inv_n = f32(1.0 / (_B * _C))

    selfhT = jnp.zeros((_H, _B), f32)
    for i in range(_NSTEP):
        reTi = reT[:, i * _B:(i + 1) * _B]
        ecqTi = ecqT[:, i * _B:(i + 1) * _B]
        adjB = adj_all[i * _B:(i + 1) * _B, :]
        revB = rev_all[i * _B:(i + 1) * _B, :]

        ecv = ec_s[...]
        shT = jnp.concatenate([selfhT, reTi], axis=0)
        s01T = dot(WA[...], shT) + b01[...]
        d01T = dot(WV[...], reTi - ecqTi)

        h1 = jnp.maximum(dot(fsW1[...], shT) + fsb1[...], 0.0)
        h2 = jnp.maximum(dot(fsW2[...], h1) + fsb2[...], 0.0)
        mu = jnp.mean(h2, axis=1, keepdims=True)
        va = jnp.mean((h2 - mu) * (h2 - mu), axis=1, keepdims=True)
        sfT = (h2 - mu) * lax.rsqrt(va + 1e-5) * fsg[...] + fsbt[...]

        def _l2(b, carry):
            ssum, ssq = carry
            X = ht_s[pl.ds(b * _H, _H), :]
            ohc = (bcol == b).astype(f32)
            s_c = dot(s01T, ohc)
            d_c = dot(d01T, ohc)
            mq = (lane == qs[b, i]).astype(f32)
            zT = dot(WU[...], X) + s_c + ecv + d_c * mq
            uT = jnp.maximum(dot(W2T[...], jnp.maximum(zT, 0.0))
                             + b2c[...], 0.0) * valid
            u_cache[pl.ds(b * 2 * _H, 2 * _H), :] = uT
            return (ssum + jnp.sum(uT, axis=1, keepdims=True),
                    ssq + jnp.sum(uT * uT, axis=1, keepdims=True))

        ssum, ssq = lax.fori_loop(
            0, _B, _l2,
            (jnp.zeros((2 * _H, 1), f32), jnp.zeros((2 * _H, 1), f32)))
        mu01 = ssum * inv_n
        var01 = ssq * inv_n - mu01 * mu01
        sc01 = g01[...] * lax.rsqrt(var01 + 1e-5)
        sh01 = bt01[...] - mu01 * sc01

        def _l3(b, carry):
            pacc, snext = carry
            X = ht_s[pl.ds(b * _H, _H), :]
            uT = u_cache[pl.ds(b * 2 * _H, 2 * _H), :]
            ohc = (bcol == b).astype(f32)
            mq = (lane == qs[b, i]).astype(f32)
            n01 = uT * sc01 + sh01
            n0 = n01[:_H, :]
            n1 = n01[_H:, :]
            r_adj = adj_all[pl.ds(i * _B + b, 1), :]
            r_rev = rev_all[pl.ds(i * _B + b, 1), :]
            neigh = r_adj * n0 + r_rev * n1
            sf_c = dot(sfT, ohc)
            mn = neigh + mq * (sf_c - neigh)
            eT = dot(WEAo[...], mn) + bEA[...]
            eg = jax.nn.sigmoid(eT[:_H, :])
            ad = jnp.tanh(eT[_H:, :])
            m2 = mn - eaw[...] * eg * mn + eaw[...] * ad
            XT = jnp.concatenate([m2, X], axis=0)
            G = dot(WG[...], XT) + bg[...]
            r = jax.nn.sigmoid(G[:_H, :])
            zg = jax.nn.sigmoid(G[_H:2 * _H, :])
            n = jnp.tanh(G[2 * _H:3 * _H, :] + r * G[3 * _H:, :])
            hn = (1.0 - zg) * n + zg * X
            ht_s[pl.ds(b * _H, _H), :] = hn
            mqn = (lane == qs[b, i + 1]).astype(f32)
            hsel = hn * mqn
            col = jnp.sum(hsel, axis=1, keepdims=True)
            ohr = (brow == b).astype(f32)
            snext = snext + col * ohr
            pv = jnp.sum(col * wp[...], axis=0, keepdims=True)
            return pacc + pv * ohc, snext

        pacc, selfhT = lax.fori_loop(
            0, _B, _l3,
            (jnp.zeros((_B, 1), f32), jnp.zeros((_H, _B), f32)))
        out_ref[:, i:i + 1] = jax.nn.sigmoid(pacc + bpv[...])


def _tc_call(args):
    return pl.pallas_call(
        _tc_body,
        out_shape=jax.ShapeDtypeStruct((_B, _NSTEP), jnp.float32),
        in_specs=[pl.BlockSpec(memory_space=pltpu.SMEM)]
        + [pl.BlockSpec(memory_space=pltpu.VMEM)] * 26,
        out_specs=pl.BlockSpec(memory_space=pltpu.VMEM),
        scratch_shapes=[
            pltpu.VMEM((_B * _H, _CW), jnp.float32),
            pltpu.VMEM((2 * _H, _CW), jnp.float32),
            pltpu.VMEM((_B * 2 * _H, _CW), jnp.float32),
        ],
    )(*args)


def kernel(features, questions, emb_x, emb_c, graph, fs_W1, fs_b1, fs_W2,
           fs_b2, fs_g, fs_bt, fn0_W1, fn0_b1, fn0_W2, fn0_b2, fn0_g,
           fn0_bt, fn1_W1, fn1_b1, fn1_W2, fn1_b2, fn1_g, fn1_bt, ea_w,
           ea_We, ea_be, ea_Wa, ea_ba, gru_Wih, gru_Whh, gru_bih, gru_bhh,
           Wp, bp):
    f32 = jnp.float32
    qi = questions.astype(jnp.int32)
    xi = features.astype(jnp.int32)

    qt_flat = qi[:, :_NSTEP].T.reshape(_NIDX)
    xt_flat = xi[:, :_NSTEP].T.reshape(_NIDX)
    padz = jnp.zeros((_PAD - _NIDX,), jnp.int32)
    idxg = jnp.concatenate([qt_flat, padz, qt_flat + _C, padz])
    idxe = jnp.concatenate([xt_flat, padz, qt_flat + 2 * _C, padz])

    gtab = jnp.pad(jnp.concatenate([graph, graph.T], axis=0),
                   ((0, 0), (0, _CW - _C)))
    etab = jnp.pad(jnp.concatenate([emb_x, emb_c[:_C, :]], axis=0),
                   ((0, 0), (0, _EW - _E)))

    gr_out, em_out = _sc_gather(gtab, etab, idxg, idxe)
    adj_all = gr_out[:_NIDX]
    rev_all = gr_out[_PAD:_PAD + _NIDX]
    reT = em_out[:_NIDX, :_E].T
    ecqT = em_out[_PAD:_PAD + _NIDX, :_E].T

    WA = jnp.concatenate([fn0_W1[:, :2 * _H], fn1_W1[:, :2 * _H]], axis=0)
    WU = jnp.concatenate([fn0_W1[:, 2 * _H:3 * _H],
                          fn1_W1[:, 2 * _H:3 * _H]], axis=0)
    WV = jnp.concatenate([fn0_W1[:, 3 * _H:], fn1_W1[:, 3 * _H:]], axis=0)
    b01 = jnp.concatenate([fn0_b1, fn1_b1]).reshape(2 * _H, 1)
    zH = jnp.zeros((_H, _H), f32)
    W2T = jnp.concatenate([
        jnp.concatenate([fn0_W2, zH], axis=1),
        jnp.concatenate([zH, fn1_W2], axis=1)], axis=0)
    b2c = jnp.concatenate([fn0_b2, fn1_b2]).reshape(2 * _H, 1)
    g01 = jnp.concatenate([fn0_g, fn1_g]).reshape(2 * _H, 1)
    bt01 = jnp.concatenate([fn0_bt, fn1_bt]).reshape(2 * _H, 1)
    WEAo = jnp.concatenate([ea_We, ea_Wa], axis=0)
    bEA = jnp.concatenate([ea_be, ea_ba]).reshape(2 * _H, 1)
    zH2 = jnp.zeros((_H, _H), f32)
    WG = jnp.concatenate([
        jnp.concatenate([gru_Wih[:_H], gru_Whh[:_H]], axis=1),
        jnp.concatenate([gru_Wih[_H:2 * _H], gru_Whh[_H:2 * _H]], axis=1),
        jnp.concatenate([gru_Wih[2 * _H:], zH2], axis=1),
        jnp.concatenate([zH2, gru_Whh[2 * _H:]], axis=1),
    ], axis=0)
    bg = jnp.concatenate([
        gru_bih[:2 * _H] + gru_bhh[:2 * _H],
        gru_bih[2 * _H:], gru_bhh[2 * _H:]]).reshape(4 * _H, 1)

    embcT = jnp.pad(emb_c[:_C, :].T, ((0, 0), (0, _CW - _C)))
    eaw_r = jnp.pad(ea_w, (0, _CW - _C)).reshape(1, _CW)

    args = (qi, reT, ecqT, adj_all, rev_all, embcT, eaw_r,
            WA, b01, WU, WV, W2T, b2c, g01, bt01,
            WEAo, bEA, WG, bg,
            fs_W1, fs_b1.reshape(_H, 1), fs_W2, fs_b2.reshape(_H, 1),
            fs_g.reshape(_H, 1), fs_bt.reshape(_H, 1),
            Wp.reshape(_H, 1), bp.reshape(1, 1))
    return _tc_call(args)
